# Initial kernel scaffold; baseline (speedup 1.0000x reference)
import jax
import jax.numpy as jnp
from jax.experimental import pallas as pl


def kernel(x, triangle_centers, W1, b1, W2, b2, W3, b3, Wf, bf):
    raise NotImplementedError("write your pallas kernel here")


# speedup vs baseline: 2.7936x; 2.7936x over previous
import functools

import jax
import jax.numpy as jnp
from jax import lax
from jax.experimental import pallas as pl
from jax.experimental.pallas import tpu as pltpu
from jax.experimental.pallas import tpu_sc as plsc

KNN = 32
N_REAL = 10000
NW = 32
NC, NS, NL = 2, 16, 16
NP = 10240
SENT = 10000
NPW = NP // NW
EPW = NPW * KNN
BN = 4
NB = NPW // BN
QB = 256
RB = 512
PADC = 1.0e18
BIGF = 3.0e38
INV_SQRT_K = float(1.0 / (32.0 ** 0.5))
F32 = jnp.float32
I32 = jnp.int32

@functools.cache
def _mesh():
    return plsc.VectorSubcoreMesh(core_axis_name="c", subcore_axis_name="s",
                                  num_cores=NC, num_subcores=NS)


def _knn_body(ptsT_ref, pts_ref, out_ref):
    i = pl.program_id(0)
    cx = ptsT_ref[0:1, :]
    cy = ptsT_ref[1:2, :]
    cz = ptsT_ref[2:3, :]
    csq = (cx * cx + cy * cy) + cz * cz
    qx = pts_ref[:, 0:1]
    qy = pts_ref[:, 1:2]
    qz = pts_ref[:, 2:3]
    qsq = (qx * qx + qy * qy) + qz * qz
    dot = qx * cx + qy * cy + qz * cz
    d = (qsq + csq) - 2.0 * dot

    col = lax.broadcasted_iota(I32, (QB, NP), 1)
    outcol = lax.broadcasted_iota(I32, (QB, KNN), 1)

    def body(t, carry):
        d, outb = carry
        m = jnp.min(d, axis=1, keepdims=True)
        cand = jnp.where(d == m, col, NP)
        amin = jnp.min(cand, axis=1, keepdims=True)
        outb = jnp.where(outcol == t, jnp.broadcast_to(amin, (QB, KNN)), outb)
        d = jnp.where(col == amin, BIGF, d)
        return d, outb

    _, outb = lax.fori_loop(0, KNN, body, (d, jnp.zeros((QB, KNN), I32)))
    row = i * QB + lax.broadcasted_iota(I32, (QB, 1), 0)
    out_ref[...] = jnp.where(row < N_REAL, outb, SENT)


def _knn_topk(ptsT, pts_pad):
    return pl.pallas_call(
        _knn_body,
        grid=(NP // QB,),
        in_specs=[
            pl.BlockSpec((3, NP), lambda i: (0, 0)),
            pl.BlockSpec((QB, 3), lambda i: (i, 0)),
        ],
        out_specs=pl.BlockSpec((QB, KNN), lambda i: (i, 0)),
        out_shape=jax.ShapeDtypeStruct((NP, KNN), I32),
    )(ptsT, pts_pad)


@functools.cache
def _sc_hist_fn():
    @functools.partial(
        pl.kernel,
        mesh=_mesh(),
        out_type=jax.ShapeDtypeStruct((NW, NP), F32),
        scratch_types=[
            pltpu.VMEM((EPW,), I32),
            pltpu.VMEM((NP,), F32),
        ],
        compiler_params=pltpu.CompilerParams(needs_layout_passes=False),
    )
    def _sc_hist(idx_hbm, out_hbm, idx_v, hist_v):
        wid = lax.axis_index("s") * NC + lax.axis_index("c")
        pltpu.sync_copy(idx_hbm.at[wid], idx_v)
        zero16 = jnp.zeros((NL,), F32)
        one16 = jnp.full((NL,), 1.0, F32)

        def zbody(j, _):
            hist_v[pl.ds(j * NL, NL)] = zero16
            return 0

        lax.fori_loop(0, NP // NL, zbody, 0)

        def abody(j, _):
            v = idx_v[pl.ds(j * NL, NL)]
            plsc.addupdate_scatter(hist_v, [v], one16)
            return 0

        lax.fori_loop(0, EPW // NL, abody, 0)
        pltpu.sync_copy(hist_v, out_hbm.at[wid])

    return _sc_hist


def _deg_body(h_ref, o_ref):
    s = jnp.sum(h_ref[...], axis=0, keepdims=True)
    o_ref[...] = lax.rsqrt(jnp.maximum(s, 1.0))


def _deg_w(hist):
    return pl.pallas_call(
        _deg_body,
        out_shape=jax.ShapeDtypeStruct((1, NP), F32),
    )(hist)


def _scale_body(x_ref, w_ref, o_ref):
    o_ref[...] = x_ref[...] * w_ref[...]


def _scale_rows(x_pad, wcol):
    return pl.pallas_call(
        _scale_body,
        grid=(NP // RB,),
        in_specs=[
            pl.BlockSpec((RB, 128), lambda i: (i, 0)),
            pl.BlockSpec((RB, 1), lambda i: (i, 0)),
        ],
        out_specs=pl.BlockSpec((RB, 128), lambda i: (i, 0)),
        out_shape=jax.ShapeDtypeStruct((NP, 128), F32),
    )(x_pad, wcol)


def _tree_sum(vs):
    while len(vs) > 1:
        nxt = [vs[i] + vs[i + 1] for i in range(0, len(vs) - 1, 2)]
        if len(vs) % 2:
            nxt.append(vs[-1])
        vs = nxt
    return vs[0]


@functools.cache
def _sc_gather_fn():
    @functools.partial(
        pl.kernel,
        mesh=_mesh(),
        out_type=jax.ShapeDtypeStruct((NP, 128), F32),
        scratch_types=[
            pltpu.VMEM((NB, BN * KNN), I32),
            pltpu.VMEM((BN * KNN, 128), F32),
            pltpu.VMEM((BN, 128), F32),
            pltpu.SemaphoreType.DMA,
        ],
        compiler_params=pltpu.CompilerParams(needs_layout_passes=False),
    )
    def _sc_gather(hs_hbm, idx_hbm, out_hbm, idx_v, rows_v, out_v, sem):
        wid = lax.axis_index("s") * NC + lax.axis_index("c")
        pltpu.sync_copy(idx_hbm.at[wid], idx_v)

        def body(b, _):
            pltpu.async_copy(hs_hbm.at[idx_v.at[b]], rows_v, sem).wait()
            for n in range(BN):
                for c in range(8):
                    sl = pl.ds(c * NL, NL)
                    acc = _tree_sum([rows_v[n * KNN + m, sl] for m in range(KNN)])
                    out_v[n, sl] = acc
            pltpu.sync_copy(out_v, out_hbm.at[pl.ds(wid * NPW + b * BN, BN)])
            return 0

        lax.fori_loop(0, NB, body, 0)

    return _sc_gather


def _mm_body(a_ref, w_ref, b_ref, wc_ref, o_ref):
    a = a_ref[...] * INV_SQRT_K
    m = lax.dot_general(a, w_ref[...], (((1,), (0,)), ((), ())),
                        precision=lax.Precision.HIGHEST,
                        preferred_element_type=F32)
    h = jnp.maximum(m + b_ref[...], 0.0)
    o_ref[...] = h * wc_ref[...]


def _mm_layer(agg, W, b, wcol):
    return pl.pallas_call(
        _mm_body,
        grid=(NP // RB,),
        in_specs=[
            pl.BlockSpec((RB, 128), lambda i: (i, 0)),
            pl.BlockSpec((128, 128), lambda i: (0, 0)),
            pl.BlockSpec((1, 128), lambda i: (0, 0)),
            pl.BlockSpec((RB, 1), lambda i: (i, 0)),
        ],
        out_specs=pl.BlockSpec((RB, 128), lambda i: (i, 0)),
        out_shape=jax.ShapeDtypeStruct((NP, 128), F32),
    )(agg, W, b, wcol)


def _mm_final_body(a_ref, w_ref, b_ref, wf_ref, bf_ref, o_ref):
    a = a_ref[...] * INV_SQRT_K
    m = lax.dot_general(a, w_ref[...], (((1,), (0,)), ((), ())),
                        precision=lax.Precision.HIGHEST,
                        preferred_element_type=F32)
    h = jnp.maximum(m + b_ref[...], 0.0)
    z = lax.dot_general(h, wf_ref[...], (((1,), (0,)), ((), ())),
                        precision=lax.Precision.HIGHEST,
                        preferred_element_type=F32) + bf_ref[...]
    o_ref[...] = 1.0 / (1.0 + jnp.exp(-z))


def _mm_final(agg, W, b, Wf, bf):
    return pl.pallas_call(
        _mm_final_body,
        grid=(NP // RB,),
        in_specs=[
            pl.BlockSpec((RB, 128), lambda i: (i, 0)),
            pl.BlockSpec((128, 128), lambda i: (0, 0)),
            pl.BlockSpec((1, 128), lambda i: (0, 0)),
            pl.BlockSpec((128, 1), lambda i: (0, 0)),
            pl.BlockSpec((1, 1), lambda i: (0, 0)),
        ],
        out_specs=pl.BlockSpec((RB, 1), lambda i: (i, 0)),
        out_shape=jax.ShapeDtypeStruct((NP, 1), F32),
    )(agg, W, b, Wf, bf)


def kernel(x, triangle_centers, W1, b1, W2, b2, W3, b3, Wf, bf):
    pts = triangle_centers
    pad_n = NP - N_REAL
    pts_pad = jnp.pad(pts, ((0, pad_n), (0, 0)), constant_values=PADC)
    ptsT = jnp.pad(pts.T, ((0, 0), (0, pad_n)), constant_values=PADC)
    x_pad = jnp.pad(x, ((0, pad_n), (0, 0)))

    idx = _knn_topk(ptsT, pts_pad)
    hist = _sc_hist_fn()(idx.reshape(NW, EPW))
    w1d = _deg_w(hist)
    wcol = w1d.reshape(NP, 1)

    idx3 = idx.reshape(NW, NB, BN * KNN)
    h = _scale_rows(x_pad, wcol)
    for W, b in ((W1, b1), (W2, b2)):
        agg = _sc_gather_fn()(h, idx3)
        h = _mm_layer(agg, W, b.reshape(1, 128), wcol)
    agg = _sc_gather_fn()(h, idx3)
    res = _mm_final(agg, W3, b3.reshape(1, 128), Wf, bf.reshape(1, 1))
    return res[:N_REAL, 0]
